# baseline (device time: 190418 ns/iter reference)
import jax
import jax.numpy as jnp
from jax import lax
from jax.experimental import pallas as pl
from jax.experimental.pallas import tpu as pltpu

N_DEV = 4
E = 32
CAP = 204


def _moe_main(xb, wb, route, tri):
    tok, d = xb.shape
    e_loc, _, h_dim = wb.shape
    hl = h_dim // 2
    hh = tok // 2

    def body(x_ref, w_ref, r_ref, tri_ref, out_ref,
             k_ref, xg, pst_dm2, pst_dp1_lo, pst_dm1_hi, pst_own, accbf,
             rs_r, rs_l, rcomm,
             ag_send, ag_recv, rss_r, rsr_r, rss_l, rsr_l, rt_send, rt_recv,
             credit_r, credit_l):
        my = lax.axis_index("i")
        left = lax.rem(my + N_DEV - 1, N_DEV)
        right = lax.rem(my + 1, N_DEV)
        opp = lax.rem(my + 2, N_DEV)

        barrier = pltpu.get_barrier_semaphore()
        for nbr in (left, right, opp):
            pl.semaphore_signal(barrier, inc=1, device_id=(nbr,),
                                device_id_type=pl.DeviceIdType.MESH)
        pl.semaphore_wait(barrier, 3)

        def rcopy(src, dst, ssem, rsem, dev):
            return pltpu.make_async_remote_copy(
                src_ref=src, dst_ref=dst, send_sem=ssem, recv_sem=rsem,
                device_id=(dev,), device_id_type=pl.DeviceIdType.MESH)

        rt = []
        for j, tgt in ((0, right), (1, left), (2, opp)):
            rd = rcopy(r_ref, rcomm.at[j], rt_send.at[j], rt_recv.at[j], tgt)
            rd.start()
            rt.append(rd)
        ag0 = rcopy(x_ref, xg.at[0], ag_send.at[0], ag_recv.at[0], right)
        ag0.start()
        agl = rcopy(x_ref, xg.at[2], ag_send.at[2], ag_recv.at[2], left)
        agl.start()

        for rd in rt:
            rd.wait_recv()
        my_experts = (my * e_loc
                      + lax.broadcasted_iota(jnp.int32, (1, e_loc), 1))
        srcs = [r_ref, rcomm.at[1], rcomm.at[2], rcomm.at[0]]
        slot_of_r = [3, 2, 1, 0]
        counts = []
        for r in range(N_DEV):
            m = (srcs[r][...] == my_experts).astype(jnp.float32)
            counts.append(jnp.sum(m, axis=0, keepdims=True))
        for r in range(N_DEV):
            gid_r = lax.rem(my + r, N_DEV)
            offset = jnp.zeros((1, e_loc), jnp.float32)
            for rp in range(N_DEV):
                if rp == r:
                    continue
                gid_rp = lax.rem(my + rp, N_DEV)
                wgt = jnp.where(gid_rp < gid_r, 1.0, 0.0)
                offset = offset + wgt * counts[rp]
            m = (srcs[r][...] == my_experts).astype(jnp.bfloat16)
            pref = jnp.dot(tri_ref[...], m,
                           preferred_element_type=jnp.float32)
            rank_excl = pref - m.astype(jnp.float32) + offset
            kept = (m > 0) & (rank_excl < jnp.float32(CAP))
            k_ref[slot_of_r[r]] = kept.astype(jnp.bfloat16)

        def partial_cols(x_val, kslot, c0, c1):
            acc = jnp.zeros((tok, c1 - c0), jnp.float32)
            for e in range(e_loc):
                ke = k_ref[kslot, :, e:e + 1]
                acc = acc + jnp.dot(x_val * ke, w_ref[e, :, c0:c1],
                                    preferred_element_type=jnp.float32)
            return acc

        def chain_pair(acc_slot, rs_buf, rs_slot, ssems, rsems, t, dev):
            return [rcopy(accbf.at[acc_slot, pl.ds(c * hh, hh)],
                          rs_buf.at[rs_slot, pl.ds(c * hh, hh)],
                          ssems.at[2 * t + c],
                          rsems.at[2 * rs_slot + c], dev)
                    for c in range(2)]

        ag0.wait_recv()
        ag1 = rcopy(xg.at[0], xg.at[1], ag_send.at[1], ag_recv.at[1], right)
        ag1.start()
        accbf[0] = partial_cols(xg[0], 0, 0, hl).astype(jnp.bfloat16)
        rsr0 = chain_pair(0, rs_r, 0, rss_r, rsr_r, 0, right)
        for rd in rsr0:
            rd.start()

        agl.wait_recv()
        accbf[1] = partial_cols(xg[2], 2, hl, h_dim).astype(jnp.bfloat16)
        rsl0 = chain_pair(1, rs_l, 0, rss_l, rsr_l, 0, left)
        for rd in rsl0:
            rd.start()

        pst_own[:, 0:hl] = partial_cols(x_ref[...], N_DEV - 1, 0, hl) \
            .astype(jnp.bfloat16)

        def step(prev_pair, acc_slot, rs_buf, prev_slot, pst_val, ssems,
                 rsems, t, dev, credit=None):
            new_slot = prev_slot ^ 1 if t == 1 else 0
            cur = chain_pair(acc_slot, rs_buf, new_slot, ssems, rsems, t, dev)
            for c in range(2):
                r0 = c * hh
                prev_pair[c].wait_recv()
                prev_pair[c].wait_send()
                if c == 0 and credit is not None:
                    pl.semaphore_wait(credit, 1)
                accbf[acc_slot, r0:r0 + hh] = \
                    rs_buf[prev_slot, r0:r0 + hh] + pst_val[r0:r0 + hh]
                cur[c].start()
            return cur

        ag1.wait_recv()
        pst_dm2[:, 0:hl] = partial_cols(xg[1], 1, 0, hl).astype(jnp.bfloat16)
        rsr1 = step(rsr0, 0, rs_r, 0, pst_dm2[:, 0:hl], rss_r, rsr_r,
                    1, right)
        pl.semaphore_signal(credit_r, inc=1, device_id=(left,),
                            device_id_type=pl.DeviceIdType.MESH)

        pst_dm2[:, hl:h_dim] = partial_cols(xg[1], 1, hl, h_dim) \
            .astype(jnp.bfloat16)
        rsl1 = step(rsl0, 1, rs_l, 0, pst_dm2[:, hl:h_dim], rss_l, rsr_l,
                    1, left)
        pl.semaphore_signal(credit_l, inc=1, device_id=(right,),
                            device_id_type=pl.DeviceIdType.MESH)

        pst_dp1_lo[...] = partial_cols(xg[2], 2, 0, hl).astype(jnp.bfloat16)
        rsr2 = step(rsr1, 0, rs_r, 1, pst_dp1_lo[...], rss_r, rsr_r,
                    2, right, credit=credit_r)

        pst_dm1_hi[...] = partial_cols(xg[0], 0, hl, h_dim).astype(jnp.bfloat16)
        rsl2 = step(rsl1, 1, rs_l, 1, pst_dm1_hi[...], rss_l, rsr_l,
                    2, left, credit=credit_l)

        pst_own[:, hl:h_dim] = partial_cols(x_ref[...], N_DEV - 1, hl, h_dim) \
            .astype(jnp.bfloat16)
        for c in range(2):
            r0 = c * hh
            rsr2[c].wait_recv()
            out_ref[r0:r0 + hh, 0:hl] = \
                (rs_r[0, r0:r0 + hh] + pst_own[r0:r0 + hh, 0:hl]) \
                .astype(jnp.float32)
        for c in range(2):
            r0 = c * hh
            rsl2[c].wait_recv()
            out_ref[r0:r0 + hh, hl:h_dim] = \
                (rs_l[0, r0:r0 + hh] + pst_own[r0:r0 + hh, hl:h_dim]) \
                .astype(jnp.float32)

        for rd in rsr2 + rsl2:
            rd.wait_send()
        ag0.wait_send()
        ag1.wait_send()
        agl.wait_send()
        for rd in rt:
            rd.wait_send()

    return pl.pallas_call(
        body,
        out_shape=jax.ShapeDtypeStruct((tok, h_dim), jnp.float32),
        in_specs=[pl.BlockSpec(memory_space=pltpu.VMEM)] * 4,
        out_specs=pl.BlockSpec(memory_space=pltpu.VMEM),
        scratch_shapes=[
            pltpu.VMEM((N_DEV, tok, e_loc), jnp.bfloat16),
            pltpu.VMEM((3, tok, d), jnp.bfloat16),
            pltpu.VMEM((tok, h_dim), jnp.bfloat16),
            pltpu.VMEM((tok, hl), jnp.bfloat16),
            pltpu.VMEM((tok, hl), jnp.bfloat16),
            pltpu.VMEM((tok, h_dim), jnp.bfloat16),
            pltpu.VMEM((2, tok, hl), jnp.bfloat16),
            pltpu.VMEM((2, tok, hl), jnp.bfloat16),
            pltpu.VMEM((2, tok, hl), jnp.bfloat16),
            pltpu.VMEM((3, tok, 1), jnp.int32),
            pltpu.SemaphoreType.DMA((3,)),
            pltpu.SemaphoreType.DMA((3,)),
            pltpu.SemaphoreType.DMA((6,)),
            pltpu.SemaphoreType.DMA((4,)),
            pltpu.SemaphoreType.DMA((6,)),
            pltpu.SemaphoreType.DMA((4,)),
            pltpu.SemaphoreType.DMA((3,)),
            pltpu.SemaphoreType.DMA((3,)),
            pltpu.SemaphoreType.REGULAR,
            pltpu.SemaphoreType.REGULAR,
        ],
        compiler_params=pltpu.CompilerParams(
            collective_id=0, vmem_limit_bytes=110 * 1024 * 1024),
    )(xb, wb, route, tri)


def kernel(x, router_W, route_idx, expert_W):
    del router_W
    tok = x.shape[0]
    xb = x.astype(jnp.bfloat16)
    wb = expert_W.astype(jnp.bfloat16)
    tri = (lax.broadcasted_iota(jnp.int32, (tok, tok), 0)
           >= lax.broadcasted_iota(jnp.int32, (tok, tok), 1)) \
        .astype(jnp.bfloat16)
    return _moe_main(xb, wb, route_idx, tri)


# device time: 185089 ns/iter; 1.0288x vs baseline; 1.0288x over previous
import jax
import jax.numpy as jnp
from jax import lax
from jax.experimental import pallas as pl
from jax.experimental.pallas import tpu as pltpu

N_DEV = 4
E = 32
CAP = 204


def _ag_route(route_shard):
    tok = route_shard.shape[0]

    def body(r_ref, out_ref, comm, send_sems, recv_sems):
        my = lax.axis_index("i")
        left = lax.rem(my + N_DEV - 1, N_DEV)
        right = lax.rem(my + 1, N_DEV)
        opp = lax.rem(my + 2, N_DEV)

        barrier = pltpu.get_barrier_semaphore()
        for nbr in (left, right):
            pl.semaphore_signal(barrier, inc=1, device_id=(nbr,),
                                device_id_type=pl.DeviceIdType.MESH)
        pl.semaphore_wait(barrier, 2)

        rdmas = []
        for j, tgt in ((0, right), (1, left), (2, opp)):
            rdma = pltpu.make_async_remote_copy(
                src_ref=r_ref,
                dst_ref=comm.at[j],
                send_sem=send_sems.at[j],
                recv_sem=recv_sems.at[j],
                device_id=(tgt,),
                device_id_type=pl.DeviceIdType.MESH,
            )
            rdma.start()
            rdmas.append(rdma)

        out_ref[pl.ds(my, 1)] = r_ref[...][None]
        for j, org in ((0, left), (1, right), (2, opp)):
            rdmas[j].wait_recv()
            out_ref[pl.ds(org, 1)] = comm[j][None]
        for rdma in rdmas:
            rdma.wait_send()

    return pl.pallas_call(
        body,
        out_shape=jax.ShapeDtypeStruct((N_DEV, tok, 1), jnp.int32),
        in_specs=[pl.BlockSpec(memory_space=pltpu.VMEM)],
        out_specs=pl.BlockSpec(memory_space=pltpu.VMEM),
        scratch_shapes=[
            pltpu.VMEM((N_DEV - 1, tok, 1), jnp.int32),
            pltpu.SemaphoreType.DMA((N_DEV - 1,)),
            pltpu.SemaphoreType.DMA((N_DEV - 1,)),
        ],
        compiler_params=pltpu.CompilerParams(collective_id=0),
    )(route_shard)


def _keep_masks(routeg, my):
    n_tok = routeg.shape[0] * routeg.shape[1]
    e_loc = E // N_DEV
    r = routeg.reshape(n_tok)
    my_experts = my * e_loc + jnp.arange(e_loc, dtype=r.dtype)
    oh = (r[:, None] == my_experts[None, :]).astype(jnp.float32)
    g = oh.reshape(64, n_tok // 64, e_loc)
    w = g.shape[1]
    m_in = (jnp.arange(w)[:, None] >= jnp.arange(w)[None, :]).astype(jnp.float32)
    pref = jnp.einsum("ij,gje->gie", m_in, g,
                      preferred_element_type=jnp.float32)
    tot = pref[:, -1, :]
    m_ex = (jnp.arange(64)[:, None] > jnp.arange(64)[None, :]).astype(jnp.float32)
    gpre = jnp.dot(m_ex, tot, preferred_element_type=jnp.float32)
    rank_excl = pref - g + gpre[:, None, :]
    kept = (g > 0.5) & (rank_excl < jnp.float32(CAP))
    blocks = kept.astype(jnp.bfloat16).reshape(N_DEV, n_tok // N_DEV, e_loc)
    return jnp.roll(blocks[::-1], my, axis=0)


def _moe_main(xb, wb, kept):
    tok, d = xb.shape
    e_loc, _, h_dim = wb.shape
    hl = h_dim // 2
    hh = tok // 2

    def body(x_ref, w_ref, k_ref, out_ref,
             xg, pst_dm2, pst_dp1_lo, pst_dm1_hi, pst_own, accbf, rs_r, rs_l,
             ag_send, ag_recv, rss_r, rsr_r, rss_l, rsr_l,
             credit_r, credit_l):
        my = lax.axis_index("i")
        left = lax.rem(my + N_DEV - 1, N_DEV)
        right = lax.rem(my + 1, N_DEV)

        barrier = pltpu.get_barrier_semaphore()
        for nbr in (left, right):
            pl.semaphore_signal(barrier, inc=1, device_id=(nbr,),
                                device_id_type=pl.DeviceIdType.MESH)
        pl.semaphore_wait(barrier, 2)

        def rcopy(src, dst, ssem, rsem, dev):
            return pltpu.make_async_remote_copy(
                src_ref=src, dst_ref=dst, send_sem=ssem, recv_sem=rsem,
                device_id=(dev,), device_id_type=pl.DeviceIdType.MESH)

        def partial_cols(x_val, kslot, c0, c1):
            acc = jnp.zeros((tok, c1 - c0), jnp.float32)
            for e in range(e_loc):
                ke = k_ref[kslot, :, e:e + 1]
                acc = acc + jnp.dot(x_val * ke, w_ref[e, :, c0:c1],
                                    preferred_element_type=jnp.float32)
            return acc

        ag0 = rcopy(x_ref, xg.at[0], ag_send.at[0], ag_recv.at[0], right)
        ag0.start()
        agl = rcopy(x_ref, xg.at[2], ag_send.at[2], ag_recv.at[2], left)
        agl.start()

        ag0.wait_recv()
        ag1 = rcopy(xg.at[0], xg.at[1], ag_send.at[1], ag_recv.at[1], right)
        ag1.start()

        def chain_pair(acc_slot, rs_buf, rs_slot, ssems, rsems, t, dev):
            return [rcopy(accbf.at[acc_slot, pl.ds(c * hh, hh)],
                          rs_buf.at[rs_slot, pl.ds(c * hh, hh)],
                          ssems.at[2 * t + c],
                          rsems.at[2 * rs_slot + c], dev)
                    for c in range(2)]

        accbf[0] = partial_cols(xg[0], 0, 0, hl).astype(jnp.bfloat16)
        rsr0 = chain_pair(0, rs_r, 0, rss_r, rsr_r, 0, right)
        for rd in rsr0:
            rd.start()

        agl.wait_recv()
        accbf[1] = partial_cols(xg[2], 2, hl, h_dim).astype(jnp.bfloat16)
        rsl0 = chain_pair(1, rs_l, 0, rss_l, rsr_l, 0, left)
        for rd in rsl0:
            rd.start()

        pst_own[:, 0:hl] = partial_cols(x_ref[...], N_DEV - 1, 0, hl) \
            .astype(jnp.bfloat16)

        def step(prev_pair, acc_slot, rs_buf, prev_slot, pst_val, ssems,
                 rsems, t, dev, credit=None):
            new_slot = prev_slot ^ 1 if t == 1 else 0
            cur = chain_pair(acc_slot, rs_buf, new_slot, ssems, rsems, t, dev)
            for c in range(2):
                r0 = c * hh
                prev_pair[c].wait_recv()
                prev_pair[c].wait_send()
                if c == 0 and credit is not None:
                    pl.semaphore_wait(credit, 1)
                accbf[acc_slot, r0:r0 + hh] = \
                    rs_buf[prev_slot, r0:r0 + hh] + pst_val[r0:r0 + hh]
                cur[c].start()
            return cur

        ag1.wait_recv()
        pst_dm2[:, 0:hl] = partial_cols(xg[1], 1, 0, hl).astype(jnp.bfloat16)
        rsr1 = step(rsr0, 0, rs_r, 0, pst_dm2[:, 0:hl], rss_r, rsr_r,
                    1, right)
        pl.semaphore_signal(credit_r, inc=1, device_id=(left,),
                            device_id_type=pl.DeviceIdType.MESH)

        pst_dm2[:, hl:h_dim] = partial_cols(xg[1], 1, hl, h_dim) \
            .astype(jnp.bfloat16)
        rsl1 = step(rsl0, 1, rs_l, 0, pst_dm2[:, hl:h_dim], rss_l, rsr_l,
                    1, left)
        pl.semaphore_signal(credit_l, inc=1, device_id=(right,),
                            device_id_type=pl.DeviceIdType.MESH)

        pst_dp1_lo[...] = partial_cols(xg[2], 2, 0, hl).astype(jnp.bfloat16)
        rsr2 = step(rsr1, 0, rs_r, 1, pst_dp1_lo[...], rss_r, rsr_r,
                    2, right, credit=credit_r)

        pst_dm1_hi[...] = partial_cols(xg[0], 0, hl, h_dim).astype(jnp.bfloat16)
        rsl2 = step(rsl1, 1, rs_l, 1, pst_dm1_hi[...], rss_l, rsr_l,
                    2, left, credit=credit_l)

        pst_own[:, hl:h_dim] = partial_cols(x_ref[...], N_DEV - 1, hl, h_dim) \
            .astype(jnp.bfloat16)
        for c in range(2):
            r0 = c * hh
            rsr2[c].wait_recv()
            out_ref[r0:r0 + hh, 0:hl] = \
                (rs_r[0, r0:r0 + hh] + pst_own[r0:r0 + hh, 0:hl]) \
                .astype(jnp.float32)
        for c in range(2):
            r0 = c * hh
            rsl2[c].wait_recv()
            out_ref[r0:r0 + hh, hl:h_dim] = \
                (rs_l[0, r0:r0 + hh] + pst_own[r0:r0 + hh, hl:h_dim]) \
                .astype(jnp.float32)

        for rd in rsr2 + rsl2:
            rd.wait_send()
        ag0.wait_send()
        ag1.wait_send()
        agl.wait_send()

    return pl.pallas_call(
        body,
        out_shape=jax.ShapeDtypeStruct((tok, h_dim), jnp.float32),
        in_specs=[pl.BlockSpec(memory_space=pltpu.VMEM)] * 3,
        out_specs=pl.BlockSpec(memory_space=pltpu.VMEM),
        scratch_shapes=[
            pltpu.VMEM((3, tok, d), jnp.bfloat16),
            pltpu.VMEM((tok, h_dim), jnp.bfloat16),
            pltpu.VMEM((tok, hl), jnp.bfloat16),
            pltpu.VMEM((tok, hl), jnp.bfloat16),
            pltpu.VMEM((tok, h_dim), jnp.bfloat16),
            pltpu.VMEM((2, tok, hl), jnp.bfloat16),
            pltpu.VMEM((2, tok, hl), jnp.bfloat16),
            pltpu.VMEM((2, tok, hl), jnp.bfloat16),
            pltpu.SemaphoreType.DMA((3,)),
            pltpu.SemaphoreType.DMA((3,)),
            pltpu.SemaphoreType.DMA((6,)),
            pltpu.SemaphoreType.DMA((4,)),
            pltpu.SemaphoreType.DMA((6,)),
            pltpu.SemaphoreType.DMA((4,)),
            pltpu.SemaphoreType.REGULAR,
            pltpu.SemaphoreType.REGULAR,
        ],
        compiler_params=pltpu.CompilerParams(
            collective_id=1, vmem_limit_bytes=100 * 1024 * 1024),
    )(xb, wb, kept)


def kernel(x, router_W, route_idx, expert_W):
    del router_W
    my = lax.axis_index("i")
    routeg = _ag_route(route_idx)
    kept = _keep_masks(routeg, my)
    xb = x.astype(jnp.bfloat16)
    wb = expert_W.astype(jnp.bfloat16)
    return _moe_main(xb, wb, kept)
